# baseline (speedup 1.0000x reference)
import functools

import jax
import jax.numpy as jnp
from jax import lax
from jax.experimental import pallas as pl
from jax.experimental.pallas import tpu as pltpu
from jax.experimental.pallas import tpu_sc as plsc

D = 64
SCALE = float(D) ** 0.5
L = 16
NC = 2
NS = 16
NW = NC * NS
CHUNK = 512


@functools.lru_cache(maxsize=None)
def _make_sc_lookup(B, V):
    assert B % (NW * CHUNK) == 0
    b_per_w = B // NW
    n_chunks = b_per_w // CHUNK
    mesh = plsc.VectorSubcoreMesh(core_axis_name="c", subcore_axis_name="s")

    @functools.partial(
        pl.kernel,
        mesh=mesh,
        out_type=jax.ShapeDtypeStruct((B, D), jnp.float32),
        scratch_types=[
            pltpu.VMEM((CHUNK,), jnp.int32),
            pltpu.VMEM((CHUNK, D), jnp.float32),
            pltpu.SemaphoreType.DMA,
        ],
        compiler_params=pltpu.CompilerParams(use_tc_tiling_on_sc=False),
    )
    def sc_lookup(x_hbm, table_hbm, out_hbm, idx_v, rows_v, gsem):
        wid = lax.axis_index("s") * NC + lax.axis_index("c")
        base = wid * b_per_w

        def chunk_body(c, carry):
            off = base + c * CHUNK
            pltpu.sync_copy(x_hbm.at[pl.ds(off, CHUNK)], idx_v)
            pltpu.async_copy(table_hbm.at[idx_v], rows_v, gsem).wait()

            def scale_body(i, carry2):
                for j in range(D // L):
                    sl = pl.ds(j * L, L)
                    rows_v[i, sl] = rows_v[i, sl] * SCALE
                return carry2

            lax.fori_loop(0, CHUNK, scale_body, 0, unroll=4)
            pltpu.sync_copy(rows_v, out_hbm.at[pl.ds(off, CHUNK)])
            return carry

        lax.fori_loop(0, n_chunks, chunk_body, 0)

    return sc_lookup


def kernel(x, table):
    xf = x.reshape(-1).astype(jnp.int32)
    out = _make_sc_lookup(xf.shape[0], table.shape[0])(xf, table)
    return out.reshape(x.shape + (D,))


# speedup vs baseline: 1.0907x; 1.0907x over previous
import functools

import jax
import jax.numpy as jnp
from jax import lax
from jax.experimental import pallas as pl
from jax.experimental.pallas import tpu as pltpu
from jax.experimental.pallas import tpu_sc as plsc

D = 64
SCALE = float(D) ** 0.5
L = 16
NC = 2
NS = 16
NW = NC * NS
CHUNK = 640


@functools.lru_cache(maxsize=None)
def _make_sc_lookup(B, V):
    assert B % (NW * CHUNK) == 0
    b_per_w = B // NW
    n_chunks = b_per_w // CHUNK
    mesh = plsc.VectorSubcoreMesh(core_axis_name="c", subcore_axis_name="s")

    @functools.partial(
        pl.kernel,
        mesh=mesh,
        out_type=jax.ShapeDtypeStruct((B, D), jnp.float32),
        scratch_types=[
            pltpu.VMEM((b_per_w,), jnp.int32),
            pltpu.VMEM((CHUNK, D), jnp.float32),
            pltpu.VMEM((CHUNK, D), jnp.float32),
            pltpu.SemaphoreType.DMA,
            pltpu.SemaphoreType.DMA,
            pltpu.SemaphoreType.DMA,
            pltpu.SemaphoreType.DMA,
        ],
        compiler_params=pltpu.CompilerParams(use_tc_tiling_on_sc=False),
    )
    def sc_lookup(x_hbm, table_hbm, out_hbm, idx_v, rows0, rows1,
                  g0, g1, s0, s1):
        wid = lax.axis_index("s") * NC + lax.axis_index("c")
        base = wid * b_per_w
        rows = (rows0, rows1)
        gsem = (g0, g1)
        ssem = (s0, s1)

        pltpu.sync_copy(x_hbm.at[pl.ds(base, b_per_w)], idx_v)

        def gather(c, b):
            pltpu.async_copy(
                table_hbm.at[idx_v.at[pl.ds(c * CHUNK, CHUNK)]],
                rows[b], gsem[b])

        def scale(b):
            def body(i, carry):
                for j in range(D // L):
                    sl = pl.ds(j * L, L)
                    rows[b][i, sl] = rows[b][i, sl] * SCALE
                return carry
            lax.fori_loop(0, CHUNK, body, 0, unroll=8)

        def scatter(c, b):
            return pltpu.async_copy(
                rows[b], out_hbm.at[pl.ds(base + c * CHUNK, CHUNK)], ssem[b])

        def wait_gather(b):
            pltpu.make_async_copy(
                table_hbm.at[idx_v.at[pl.ds(0, CHUNK)]], rows[b],
                gsem[b]).wait()

        def wait_scatter(b):
            pltpu.make_async_copy(
                rows[b], out_hbm.at[pl.ds(base, CHUNK)], ssem[b]).wait()

        gather(0, 0)

        def outer(g, carry):
            for b in range(2):
                c = g + b
                bn = 1 - b

                @pl.when((c >= 1) & (c + 1 < n_chunks))
                def _():
                    wait_scatter(bn)

                @pl.when(c + 1 < n_chunks)
                def _():
                    gather(c + 1, bn)

                wait_gather(b)
                scale(b)
                scatter(c, b)
            return carry

        lax.fori_loop(0, n_chunks // 2, lambda g, cc: outer(g * 2, cc), 0)

        wait_scatter(0)
        wait_scatter(1)

    return sc_lookup


def kernel(x, table):
    xf = x.reshape(-1).astype(jnp.int32)
    out = _make_sc_lookup(xf.shape[0], table.shape[0])(xf, table)
    return out.reshape(x.shape + (D,))
